# Initial kernel scaffold; baseline (speedup 1.0000x reference)
import jax
import jax.numpy as jnp
from jax.experimental import pallas as pl


def kernel(x, edge_index, xyz, subgraph_id, tau, hard_, logits, lin_in_W, lin_in_b, sage_Wl, sage_bl, sage_Wr, lins_W, lins_b, ln_g, ln_b, pred_W, pred_b, mlp_W0, mlp_b0, mlp_W1, mlp_b1, mlp_W2, mlp_b2, xyz_W, xyz_b):
    raise NotImplementedError("write your pallas kernel here")


# speedup vs baseline: 3.0274x; 3.0274x over previous
import functools

import jax
import jax.numpy as jnp
from jax import lax
from jax.experimental import pallas as pl
from jax.experimental.pallas import tpu as pltpu
from jax.experimental.pallas import tpu_sc as plsc

N = 10000
E = 160000
GENE = 256
HID = 512
OUT = 32
NLAYERS = 4

NC = 2
NS = 16
CW = 128
NCH = HID // CW
EB = 128
NB = -(-E // (NS * EB))
E_PAD = NS * EB * NB
N_SP = 10112
TPT = N_SP // NS
NPT = N // NS

BN = 400
NBLK = N // BN


def _sc_mesh():
    return plsc.VectorSubcoreMesh(core_axis_name="c", subcore_axis_name="s",
                                  num_cores=NC, num_subcores=NS)


def _segsum_body(hT, srcoff, dstb, out, src_v, dst_v, rows_v, zbuf, acc, sem):
    core = lax.axis_index("c")
    tile = lax.axis_index("s")

    def _z(i, _):
        zbuf[i // 8, pl.ds((i % 8) * 16, 16)] = jnp.zeros((16,), jnp.float32)
        return 0
    lax.fori_loop(0, 8 * 8, _z, 0)

    pltpu.sync_copy(dstb.at[tile], dst_v)
    start = jnp.minimum(tile * TPT, N - TPT)

    for j in range(2):
        def _zc(r, _):
            pltpu.sync_copy(zbuf, acc.at[pl.ds(tile * TPT + r * 8, 8)])
            return 0
        lax.fori_loop(0, TPT // 8, _zc, 0)
        pltpu.sync_copy(srcoff.at[core, j, tile], src_v)
        plsc.subcore_barrier()

        def _eb(b, _):
            pltpu.async_copy(hT.at[src_v.at[b]], rows_v, sem).wait()
            pltpu.sync_copy(rows_v, acc.at[dst_v.at[b]], add=True)
            return 0
        lax.fori_loop(0, NB, _eb, 0)
        plsc.subcore_barrier()

        chunk = core * 2 + j
        pltpu.sync_copy(acc.at[pl.ds(start, TPT)],
                        out.at[pl.ds(chunk * N + start, TPT)])
        plsc.subcore_barrier()


def _degree_body(dstb, ones_hbm, out, dst_v, ones_v, zbuf, acc):
    core = lax.axis_index("c")
    tile = lax.axis_index("s")

    def _z(i, _):
        zbuf[i // 8, pl.ds((i % 8) * 16, 16)] = jnp.zeros((16,), jnp.float32)
        return 0
    lax.fori_loop(0, 8 * 8, _z, 0)
    pltpu.sync_copy(ones_hbm, ones_v)

    def _zc(r, _):
        pltpu.sync_copy(zbuf, acc.at[pl.ds(tile * TPT + r * 8, 8)])
        return 0
    lax.fori_loop(0, TPT // 8, _zc, 0)
    pltpu.sync_copy(dstb.at[tile], dst_v)
    plsc.subcore_barrier()

    def _eb(b, _):
        pltpu.sync_copy(ones_v, acc.at[dst_v.at[b]], add=True)
        return 0

    @pl.when(core == 0)
    def _():
        lax.fori_loop(0, NB, _eb, 0)
    plsc.subcore_barrier()

    start = jnp.minimum(tile * TPT, N - TPT)

    @pl.when(core == 0)
    def _():
        pltpu.sync_copy(acc.at[pl.ds(start, TPT)], out.at[pl.ds(start, TPT)])


@functools.cache
def _sc_kernels():
    segsum = pl.kernel(
        _segsum_body,
        out_type=jax.ShapeDtypeStruct((NCH * N, CW), jnp.float32),
        mesh=_sc_mesh(),
        scratch_types=[
            pltpu.VMEM((NB, EB), jnp.int32),
            pltpu.VMEM((NB, EB), jnp.int32),
            pltpu.VMEM((EB, CW), jnp.float32),
            pltpu.VMEM((8, CW), jnp.float32),
            pltpu.VMEM_SHARED((N_SP, CW), jnp.float32),
            pltpu.SemaphoreType.DMA,
        ],
    )
    degree = pl.kernel(
        _degree_body,
        out_type=jax.ShapeDtypeStruct((N, CW), jnp.float32),
        mesh=_sc_mesh(),
        scratch_types=[
            pltpu.VMEM((NB, EB), jnp.int32),
            pltpu.VMEM((EB, CW), jnp.float32),
            pltpu.VMEM((8, CW), jnp.float32),
            pltpu.VMEM_SHARED((N_SP, CW), jnp.float32),
        ],
    )
    return segsum, degree


def _segsum(hT_flat, srcoff, dstp):
    return _sc_kernels()[0](hT_flat, srcoff, dstp)


def _degree(dstp):
    return _sc_kernels()[1](dstp, jnp.ones((EB, CW), jnp.float32))


def _mask_from_logits(logits):
    m = jnp.max(logits, axis=1, keepdims=True)
    it = lax.broadcasted_iota(jnp.int32, (64, GENE), 1)
    first = jnp.min(jnp.where(logits == m, it, GENE), axis=1, keepdims=True)
    return jnp.max(jnp.where(it == first, 1.0, 0.0), axis=0, keepdims=True)


def _prelude_body(x_ref, logits_ref, w_ref, b_ref, out_ref):
    mask = _mask_from_logits(logits_ref[...])
    xm = x_ref[...] * mask
    out_ref[0] = jnp.dot(xm, w_ref[...], preferred_element_type=jnp.float32) + b_ref[0]


def _prelude_call(x, logits, w, b4):
    b4 = b4.reshape(NCH, 1, CW)
    return pl.pallas_call(
        _prelude_body,
        grid=(NBLK, NCH),
        in_specs=[
            pl.BlockSpec((BN, GENE), lambda i, c: (i, 0)),
            pl.BlockSpec((64, GENE), lambda i, c: (0, 0)),
            pl.BlockSpec((GENE, CW), lambda i, c: (0, c)),
            pl.BlockSpec((1, 1, CW), lambda i, c: (c, 0, 0)),
        ],
        out_specs=pl.BlockSpec((1, BN, CW), lambda i, c: (c, i, 0)),
        out_shape=jax.ShapeDtypeStruct((NCH, N, CW), jnp.float32),
    )(x, logits, w, b4)


def _base_body(x_ref, logits_ref, xyz_ref, w0, b0, w1, b1, w2, b2, wx, bx, out_ref):
    mask = _mask_from_logits(logits_ref[...])
    xm = x_ref[...] * mask
    h = jnp.maximum(jnp.dot(xm, w0[...], preferred_element_type=jnp.float32) + b0[...], 0.0)
    h = jnp.maximum(jnp.dot(h, w1[...], preferred_element_type=jnp.float32) + b1[...], 0.0)
    r = jnp.dot(h, w2[...], preferred_element_type=jnp.float32) + b2[...]
    xo = jnp.dot(xyz_ref[...], wx[...], preferred_element_type=jnp.float32) + bx[...]
    out_ref[...] = r + xo


def _base_call(x, logits, xyz, w0, b0, w1, b1, w2, b2, wx, bx):
    full = lambda a, b: pl.BlockSpec((a, b), lambda i: (0, 0))
    return pl.pallas_call(
        _base_body,
        grid=(NBLK,),
        in_specs=[
            pl.BlockSpec((BN, GENE), lambda i: (i, 0)),
            full(64, GENE),
            pl.BlockSpec((BN, 3), lambda i: (i, 0)),
            full(GENE, 128), full(1, 128),
            full(128, 128), full(1, 128),
            full(128, OUT), full(1, OUT),
            full(3, OUT), full(1, OUT),
        ],
        out_specs=pl.BlockSpec((BN, OUT), lambda i: (i, 0)),
        out_shape=jax.ShapeDtypeStruct((N, OUT), jnp.float32),
    )(x, logits, xyz, w0, b0.reshape(1, 128), w1, b1.reshape(1, 128),
      w2, b2.reshape(1, OUT), wx, bx.reshape(1, OUT))


def _layer_body(aggr_ref, h_ref, xf_ref, deg_ref, wl_ref, wc_ref, bc_ref,
                g_ref, b_ref, hout_ref, xfout_ref):
    inv = 1.0 / jnp.maximum(deg_ref[:, 0:1], 1.0)
    aggr = jnp.concatenate([aggr_ref[c] for c in range(NCH)], axis=1) * inv
    h = jnp.concatenate([h_ref[c] for c in range(NCH)], axis=1)
    z = (jnp.dot(aggr, wl_ref[...], preferred_element_type=jnp.float32)
         + jnp.dot(h, wc_ref[...], preferred_element_type=jnp.float32)
         + bc_ref[...])
    mu = jnp.mean(z, axis=1, keepdims=True)
    d = z - mu
    var = jnp.mean(d * d, axis=1, keepdims=True)
    hn = jnp.maximum(d * lax.rsqrt(var + 1e-5) * g_ref[...] + b_ref[...], 0.0)
    for c in range(NCH):
        hc = hn[:, c * CW:(c + 1) * CW]
        hout_ref[c] = hc
        xfout_ref[c] = xf_ref[c] + hc


def _layer_call(aggr, h, xf, degp, wl, wc, bc, g, b):
    blk3 = pl.BlockSpec((NCH, BN, CW), lambda i: (0, i, 0))
    full = lambda a, bb: pl.BlockSpec((a, bb), lambda i: (0, 0))
    return pl.pallas_call(
        _layer_body,
        grid=(NBLK,),
        in_specs=[
            blk3, blk3,
            blk3,
            pl.BlockSpec((BN, CW), lambda i: (i, 0)),
            full(HID, HID), full(HID, HID), full(1, HID),
            full(1, HID), full(1, HID),
        ],
        out_specs=[blk3, blk3],
        out_shape=[jax.ShapeDtypeStruct((NCH, N, CW), jnp.float32),
                   jax.ShapeDtypeStruct((NCH, N, CW), jnp.float32)],
    )(aggr, h, xf, degp, wl, wc, bc, g, b)


def _final_body(xf_ref, base_ref, w_ref, b_ref, out_ref):
    acc = base_ref[...] + b_ref[...]
    for c in range(NCH):
        acc = acc + jnp.dot(xf_ref[c], w_ref[pl.ds(c * CW, CW), :],
                            preferred_element_type=jnp.float32)
    out_ref[...] = acc


def _final_call(xf, base, w, b):
    return pl.pallas_call(
        _final_body,
        grid=(NBLK,),
        in_specs=[
            pl.BlockSpec((NCH, BN, CW), lambda i: (0, i, 0)),
            pl.BlockSpec((BN, OUT), lambda i: (i, 0)),
            pl.BlockSpec((HID, OUT), lambda i: (0, 0)),
            pl.BlockSpec((1, OUT), lambda i: (0, 0)),
        ],
        out_specs=pl.BlockSpec((BN, OUT), lambda i: (i, 0)),
        out_shape=jax.ShapeDtypeStruct((N, OUT), jnp.float32),
    )(xf, base, w, b.reshape(1, OUT))


def kernel(x, edge_index, xyz, subgraph_id, tau, hard_, logits,
           lin_in_W, lin_in_b, sage_Wl, sage_bl, sage_Wr, lins_W, lins_b,
           ln_g, ln_b, pred_W, pred_b, mlp_W0, mlp_b0, mlp_W1, mlp_b1,
           mlp_W2, mlp_b2, xyz_W, xyz_b):
    src = edge_index[0].astype(jnp.int32)
    dst = edge_index[1].astype(jnp.int32)
    pad = E_PAD - E
    srcp = jnp.concatenate([src, jnp.zeros((pad,), jnp.int32)]).reshape(NS, NB, EB)
    dstp = jnp.concatenate([dst, jnp.full((pad,), N, jnp.int32)]).reshape(NS, NB, EB)
    srcoff = (srcp[None] + (jnp.arange(NCH, dtype=jnp.int32) * N)[:, None, None, None])
    srcoff = srcoff.reshape(NC, 2, NS, NB, EB)

    degp = _degree(dstp)
    h = _prelude_call(x, logits, lin_in_W, lin_in_b.reshape(NCH, CW))
    base = _base_call(x, logits, xyz, mlp_W0, mlp_b0, mlp_W1, mlp_b1,
                      mlp_W2, mlp_b2, xyz_W, xyz_b)

    xf = jnp.zeros((NCH, N, CW), jnp.float32)
    for i in range(NLAYERS):
        aggr = _segsum(h.reshape(NCH * N, CW), srcoff, dstp).reshape(NCH, N, CW)
        wc = sage_Wr[i] + lins_W[i]
        bc = (sage_bl[i] + lins_b[i]).reshape(1, HID)
        h, xf = _layer_call(aggr, h, xf, degp, sage_Wl[i], wc, bc,
                            ln_g[i].reshape(1, HID), ln_b[i].reshape(1, HID))

    return _final_call(xf, base, pred_W, pred_b)
